# Initial kernel scaffold; baseline (speedup 1.0000x reference)
import jax
import jax.numpy as jnp
from jax.experimental import pallas as pl


def kernel(desc0, desc1, kpts0, kpts1, ln0_g, ln0_b, Wq, bq, Wk, bk, Wv, bv, Wm, bm, ln1_g, ln1_b, W1, b1, W2, b2, ln2_g, ln2_b):
    raise NotImplementedError("write your pallas kernel here")


# speedup vs baseline: 1.8384x; 1.8384x over previous
import functools

import jax
import jax.numpy as jnp
from jax.experimental import pallas as pl
from jax.experimental.pallas import tpu as pltpu

D_MODEL = 256
NHEAD = 4
DH = D_MODEL // NHEAD
N_ROWS = 4096
BLOCK_M = 512
SCALE = 1.0 / (DH ** 0.5)


def _ln(x, g, b):
    m = jnp.mean(x, axis=-1, keepdims=True)
    v = jnp.mean((x - m) ** 2, axis=-1, keepdims=True)
    return (x - m) * jax.lax.rsqrt(v + 1e-5) * g + b


def _encoder_kernel(xq_ref, xs_ref, ln0g, ln0b, wq, bq, wk, bk, wv, bv,
                    wm, bm, ln1g, ln1b, w1, b1, w2, b2, ln2g, ln2b,
                    out_ref, k_s, v_s):
    i = pl.program_id(1)

    @pl.when(i == 0)
    def _compute_kv():
        sn = _ln(xs_ref[0], ln0g[...], ln0b[...])
        k_s[...] = jnp.dot(sn, wk[...], preferred_element_type=jnp.float32) + bk[...]
        v_s[...] = jnp.dot(sn, wv[...], preferred_element_type=jnp.float32) + bv[...]

    xn = _ln(xq_ref[0], ln0g[...], ln0b[...])
    q = jnp.dot(xn, wq[...], preferred_element_type=jnp.float32) + bq[...]

    msgs = []
    for h in range(NHEAD):
        qh = q[:, h * DH:(h + 1) * DH]
        kh = k_s[:, h * DH:(h + 1) * DH]
        vh = v_s[:, h * DH:(h + 1) * DH]
        s = jax.lax.dot_general(
            qh, kh, (((1,), (1,)), ((), ())),
            preferred_element_type=jnp.float32) * SCALE
        s = s - jnp.max(s, axis=-1, keepdims=True)
        e = jnp.exp(s)
        p = e / jnp.sum(e, axis=-1, keepdims=True)
        msgs.append(jnp.dot(p, vh, preferred_element_type=jnp.float32))
    msg = jnp.concatenate(msgs, axis=-1)

    m2 = _ln(jnp.dot(msg, wm[...], preferred_element_type=jnp.float32) + bm[...],
             ln1g[...], ln1b[...])
    hcat = jnp.concatenate([xn, m2], axis=-1)
    h1 = jnp.dot(hcat, w1[...], preferred_element_type=jnp.float32) + b1[...]
    hmid = 0.5 * h1 * (1.0 + jax.lax.erf(h1 * (2.0 ** -0.5)))
    y = jnp.dot(hmid, w2[...], preferred_element_type=jnp.float32) + b2[...]
    out_ref[0] = _ln(y, ln2g[...], ln2b[...])


@functools.partial(jax.jit, static_argnames=())
def kernel(desc0, desc1, kpts0, kpts1, ln0_g, ln0_b, Wq, bq, Wk, bk, Wv, bv,
           Wm, bm, ln1_g, ln1_b, W1, b1, W2, b2, ln2_g, ln2_b):
    del kpts0, kpts1
    x0 = jnp.swapaxes(desc0[0], 0, 1)
    x1 = jnp.swapaxes(desc1[0], 0, 1)
    X = jnp.stack([x0, x1])

    def row2(a):
        return a.reshape(1, -1)

    nb = N_ROWS // BLOCK_M
    full = lambda shape: pl.BlockSpec(shape, lambda d, i: (0,) * len(shape))
    delta = pl.pallas_call(
        _encoder_kernel,
        grid=(2, nb),
        in_specs=[
            pl.BlockSpec((1, BLOCK_M, D_MODEL), lambda d, i: (d, i, 0)),
            pl.BlockSpec((1, N_ROWS, D_MODEL), lambda d, i: (1 - d, 0, 0)),
            full((1, D_MODEL)), full((1, D_MODEL)),
            full((D_MODEL, D_MODEL)), full((1, D_MODEL)),
            full((D_MODEL, D_MODEL)), full((1, D_MODEL)),
            full((D_MODEL, D_MODEL)), full((1, D_MODEL)),
            full((D_MODEL, D_MODEL)), full((1, D_MODEL)),
            full((1, D_MODEL)), full((1, D_MODEL)),
            full((2 * D_MODEL, 2 * D_MODEL)), full((1, 2 * D_MODEL)),
            full((2 * D_MODEL, D_MODEL)), full((1, D_MODEL)),
            full((1, D_MODEL)), full((1, D_MODEL)),
        ],
        out_specs=pl.BlockSpec((1, BLOCK_M, D_MODEL), lambda d, i: (d, i, 0)),
        out_shape=jax.ShapeDtypeStruct((2, N_ROWS, D_MODEL), jnp.float32),
        scratch_shapes=[
            pltpu.VMEM((N_ROWS, D_MODEL), jnp.float32),
            pltpu.VMEM((N_ROWS, D_MODEL), jnp.float32),
        ],
    )(X, X, row2(ln0_g), row2(ln0_b), Wq, row2(bq), Wk, row2(bk), Wv, row2(bv),
      Wm, row2(bm), row2(ln1_g), row2(ln1_b), W1, row2(b1), W2, row2(b2),
      row2(ln2_g), row2(ln2_b))

    desc0_out = desc0 + jnp.swapaxes(delta[0], 0, 1)[None]
    desc1_out = desc1 + jnp.swapaxes(delta[1], 0, 1)[None]
    return (desc0_out, desc1_out)
